# baseline (speedup 1.0000x reference)
import functools

import jax
import jax.numpy as jnp
from jax import lax
from jax.experimental import pallas as pl
from jax.experimental.pallas import tpu as pltpu
from jax.experimental.pallas import tpu_sc as plsc

_MAX_VAL = 2097151
_NC = 2
_NS = 16
_NW = _NC * _NS
_L = 16
_NSUB = 4


@functools.lru_cache(maxsize=None)
def _build(n):
    assert n % (_NW * 4 * _L * _NSUB) == 0, n
    e_tile = n // _NW
    e_sub = e_tile // _NSUB
    n1sub = 8
    e_sub1 = e_tile // n1sub

    mesh = plsc.VectorSubcoreMesh(
        core_axis_name="c", subcore_axis_name="s",
        num_cores=_NC, num_subcores=_NS,
    )
    cparams = pltpu.CompilerParams(needs_layout_passes=False)

    @functools.partial(
        pl.kernel,
        out_type=jax.ShapeDtypeStruct((_NW, _L), jnp.int32),
        mesh=mesh,
        scratch_types=[
            pltpu.VMEM((2, e_sub1), jnp.int32),
            pltpu.VMEM((_L,), jnp.int32),
            pltpu.SemaphoreType.DMA,
            pltpu.SemaphoreType.DMA,
        ],
        compiler_params=cparams,
    )
    def _sums_kernel(mask_hbm, out_hbm, buf, outv, isem0, isem1):
        wid = lax.axis_index("c") * _NS + lax.axis_index("s")
        base = wid * e_tile
        isems = (isem0, isem1)

        def start_in(sub):
            cur = sub % 2
            return pltpu.async_copy(
                mask_hbm.at[pl.ds(base + sub * e_sub1, e_sub1)],
                buf.at[cur], isems[cur])

        in_h = {0: start_in(0)}
        acc = jnp.zeros((_L,), jnp.int32)
        for sub in range(n1sub):
            cur = sub % 2
            if sub + 1 < n1sub:
                in_h[sub + 1] = start_in(sub + 1)
            in_h.pop(sub).wait()

            def it(i, acc, cur=cur):
                a = buf[cur, pl.ds(i * 4 * _L, _L)]
                b = buf[cur, pl.ds((i * 4 + 1) * _L, _L)]
                c = buf[cur, pl.ds((i * 4 + 2) * _L, _L)]
                d = buf[cur, pl.ds((i * 4 + 3) * _L, _L)]
                return acc + ((a + b) + (c + d))

            acc = lax.fori_loop(0, e_sub1 // (4 * _L), it, acc)

        outv[...] = acc
        pltpu.sync_copy(outv, out_hbm.at[wid])

    @functools.partial(
        pl.kernel,
        out_type=jax.ShapeDtypeStruct((n,), jnp.int32),
        mesh=mesh,
        scratch_types=[
            pltpu.VMEM((e_sub,), jnp.int32),
            pltpu.VMEM((e_sub,), jnp.int32),
            pltpu.VMEM((_NW, _L), jnp.int32),
        ],
        compiler_params=cparams,
    )
    def _scan_kernel(mask_hbm, sums_hbm, out_hbm, mbuf, obuf, sums_v):
        wid = lax.axis_index("c") * _NS + lax.axis_index("s")
        base = wid * e_tile
        pltpu.sync_copy(sums_hbm, sums_v)

        def acc_row(wp, carryv):
            m = (wp < wid).astype(jnp.int32)
            return carryv + sums_v[wp] * m

        carry0 = lax.fori_loop(0, _NW, acc_row, jnp.zeros((_L,), jnp.int32))
        carry0 = jnp.sum(carry0) - 1

        def sub_body(sub, carry):
            pltpu.sync_copy(mask_hbm.at[pl.ds(base + sub * e_sub, e_sub)],
                            mbuf)

            def it(i, carry):
              for u in range(4):
                j = i * 4 + u
                v0 = mbuf[pl.ds(j * 4 * _L, _L)]
                v1 = mbuf[pl.ds((j * 4 + 1) * _L, _L)]
                v2 = mbuf[pl.ds((j * 4 + 2) * _L, _L)]
                v3 = mbuf[pl.ds((j * 4 + 3) * _L, _L)]
                packed = (v0 + (v1 << 8)) + ((v2 << 16) + (v3 << 24))
                incl = plsc.cumsum(packed)
                tv = jnp.sum(packed)
                cbef = tv * 0x01010100
                b0 = incl & 0xFF
                b1 = (incl >> 8) & 0xFF
                b2 = (incl >> 16) & 0xFF
                b3 = incl >> 24
                o0 = carry + b0
                o1 = (carry + ((cbef >> 8) & 0xFF)) + b1
                o2 = (carry + ((cbef >> 16) & 0xFF)) + b2
                o3 = (carry + (cbef >> 24)) + b3
                zero = jnp.int32(0)
                obuf[pl.ds(j * 4 * _L, _L)] = jnp.minimum(
                    jnp.maximum(o0, zero), _MAX_VAL)
                obuf[pl.ds((j * 4 + 1) * _L, _L)] = jnp.minimum(
                    jnp.maximum(o1, zero), _MAX_VAL)
                obuf[pl.ds((j * 4 + 2) * _L, _L)] = jnp.minimum(
                    jnp.maximum(o2, zero), _MAX_VAL)
                obuf[pl.ds((j * 4 + 3) * _L, _L)] = jnp.minimum(
                    jnp.maximum(o3, zero), _MAX_VAL)
                carry = carry + ((cbef >> 24) + (tv >> 24))
              return carry

            carry = lax.fori_loop(0, e_sub // (16 * _L), it, carry)
            pltpu.sync_copy(obuf, out_hbm.at[pl.ds(base + sub * e_sub, e_sub)])
            return carry

        lax.fori_loop(0, _NSUB, sub_body, carry0)

    def run(mask_i32):
        sums = _sums_kernel(mask_i32)
        return _scan_kernel(mask_i32, sums)

    return run


def kernel(mask_1d):
    n = mask_1d.shape[0]
    return _build(n)(mask_1d.astype(jnp.int32))


# speedup vs baseline: 1.0066x; 1.0066x over previous
import functools

import jax
import jax.numpy as jnp
from jax import lax
from jax.experimental import pallas as pl
from jax.experimental.pallas import tpu as pltpu
from jax.experimental.pallas import tpu_sc as plsc

_MAX_VAL = 2097151
_NC = 2
_NS = 16
_NW = _NC * _NS
_L = 16
_NSUB = 4


@functools.lru_cache(maxsize=None)
def _build(n):
    assert n % (_NW * 4 * _L * _NSUB) == 0, n
    e_tile = n // _NW
    e_sub = e_tile // _NSUB

    mesh = plsc.VectorSubcoreMesh(
        core_axis_name="c", subcore_axis_name="s",
        num_cores=_NC, num_subcores=_NS,
    )
    cparams = pltpu.CompilerParams(needs_layout_passes=False)

    @functools.partial(
        pl.kernel,
        out_type=jax.ShapeDtypeStruct((_NW, _L), jnp.int32),
        mesh=mesh,
        scratch_types=[
            pltpu.VMEM((2, e_sub), jnp.int32),
            pltpu.VMEM((_L,), jnp.int32),
            pltpu.SemaphoreType.DMA,
            pltpu.SemaphoreType.DMA,
        ],
        compiler_params=cparams,
    )
    def _sums_kernel(mask_hbm, out_hbm, buf, outv, isem0, isem1):
        wid = lax.axis_index("c") * _NS + lax.axis_index("s")
        base = wid * e_tile
        isems = (isem0, isem1)

        def start_in(sub):
            cur = sub % 2
            return pltpu.async_copy(
                mask_hbm.at[pl.ds(base + sub * e_sub, e_sub)],
                buf.at[cur], isems[cur])

        in_h = {0: start_in(0)}
        acc = jnp.zeros((_L,), jnp.int32)
        for sub in range(_NSUB):
            cur = sub % 2
            if sub + 1 < _NSUB:
                in_h[sub + 1] = start_in(sub + 1)
            in_h.pop(sub).wait()

            def it(i, acc, cur=cur):
                a = buf[cur, pl.ds(i * 4 * _L, _L)]
                b = buf[cur, pl.ds((i * 4 + 1) * _L, _L)]
                c = buf[cur, pl.ds((i * 4 + 2) * _L, _L)]
                d = buf[cur, pl.ds((i * 4 + 3) * _L, _L)]
                return acc + ((a + b) + (c + d))

            acc = lax.fori_loop(0, e_sub // (4 * _L), it, acc)

        outv[...] = acc
        pltpu.sync_copy(outv, out_hbm.at[wid])

    @functools.partial(
        pl.kernel,
        out_type=jax.ShapeDtypeStruct((n,), jnp.int32),
        mesh=mesh,
        scratch_types=[
            pltpu.VMEM((e_sub,), jnp.int32),
            pltpu.VMEM((e_sub,), jnp.int32),
            pltpu.VMEM((_NW, _L), jnp.int32),
        ],
        compiler_params=cparams,
    )
    def _scan_kernel(mask_hbm, sums_hbm, out_hbm, mbuf, obuf, sums_v):
        wid = lax.axis_index("c") * _NS + lax.axis_index("s")
        base = wid * e_tile
        pltpu.sync_copy(sums_hbm, sums_v)

        def acc_row(wp, carryv):
            m = (wp < wid).astype(jnp.int32)
            return carryv + sums_v[wp] * m

        carry0 = lax.fori_loop(0, _NW, acc_row, jnp.zeros((_L,), jnp.int32))
        carry0 = jnp.sum(carry0) - 1

        def sub_body(sub, carry):
            pltpu.sync_copy(mask_hbm.at[pl.ds(base + sub * e_sub, e_sub)],
                            mbuf)

            def it(i, carry):
              for u in range(4):
                j = i * 4 + u
                v0 = mbuf[pl.ds(j * 4 * _L, _L)]
                v1 = mbuf[pl.ds((j * 4 + 1) * _L, _L)]
                v2 = mbuf[pl.ds((j * 4 + 2) * _L, _L)]
                v3 = mbuf[pl.ds((j * 4 + 3) * _L, _L)]
                packed = (v0 + (v1 << 8)) + ((v2 << 16) + (v3 << 24))
                incl = plsc.cumsum(packed)
                tv = jnp.sum(packed)
                cbef = tv * 0x01010100
                b0 = incl & 0xFF
                b1 = (incl >> 8) & 0xFF
                b2 = (incl >> 16) & 0xFF
                b3 = incl >> 24
                o0 = carry + b0
                o1 = (carry + ((cbef >> 8) & 0xFF)) + b1
                o2 = (carry + ((cbef >> 16) & 0xFF)) + b2
                o3 = (carry + (cbef >> 24)) + b3
                zero = jnp.int32(0)
                obuf[pl.ds(j * 4 * _L, _L)] = jnp.minimum(
                    jnp.maximum(o0, zero), _MAX_VAL)
                obuf[pl.ds((j * 4 + 1) * _L, _L)] = jnp.minimum(
                    jnp.maximum(o1, zero), _MAX_VAL)
                obuf[pl.ds((j * 4 + 2) * _L, _L)] = jnp.minimum(
                    jnp.maximum(o2, zero), _MAX_VAL)
                obuf[pl.ds((j * 4 + 3) * _L, _L)] = jnp.minimum(
                    jnp.maximum(o3, zero), _MAX_VAL)
                carry = carry + ((cbef >> 24) + (tv >> 24))
              return carry

            carry = lax.fori_loop(0, e_sub // (16 * _L), it, carry)
            pltpu.sync_copy(obuf, out_hbm.at[pl.ds(base + sub * e_sub, e_sub)])
            return carry

        lax.fori_loop(0, _NSUB, sub_body, carry0)

    def run(mask_i32):
        sums = _sums_kernel(mask_i32)
        return _scan_kernel(mask_i32, sums)

    return run


def kernel(mask_1d):
    n = mask_1d.shape[0]
    return _build(n)(mask_1d.astype(jnp.int32))
